# baseline (speedup 1.0000x reference)
import functools

import jax
import jax.numpy as jnp
import numpy as np
from jax import lax
from jax.experimental import pallas as pl
from jax.experimental.pallas import tpu as pltpu
from jax.experimental.pallas import tpu_sc as plsc

B = 4
T = 2048
D = 768
NV = 8
NS_OUT = 5
NROWS_OUT = B * T * NS_OUT

NC = 2
NSUB = 16
NW = NC * NSUB
K = 32
LANES = 16
DV = D // LANES


def _rotl32(x, r):
    return ((x << np.uint32(r)) | (x >> np.uint32(32 - r))).astype(np.uint32)


def _threefry2x32(k0, k1, x0, x1):
    ks0 = np.uint32(k0)
    ks1 = np.uint32(k1)
    ks2 = np.uint32(ks0 ^ ks1 ^ np.uint32(0x1BD11BDA))
    x0 = (x0 + ks0).astype(np.uint32)
    x1 = (x1 + ks1).astype(np.uint32)
    rot0 = (13, 15, 26, 6)
    rot1 = (17, 29, 16, 24)
    ks = (ks0, ks1, ks2)
    for i in range(5):
        for r in rot0 if i % 2 == 0 else rot1:
            x0 = (x0 + x1).astype(np.uint32)
            x1 = _rotl32(x1, r)
            x1 = (x1 ^ x0).astype(np.uint32)
        x0 = (x0 + ks[(i + 1) % 3]).astype(np.uint32)
        x1 = (x1 + ks[(i + 2) % 3] + np.uint32(i + 1)).astype(np.uint32)
    return x0, x1


def _noise_constant():
    n = B * T * NV
    idx = np.arange(n, dtype=np.uint64)
    o0, o1 = _threefry2x32(0, 42, (idx >> np.uint64(32)).astype(np.uint32),
                           idx.astype(np.uint32))
    bits = (o0 ^ o1).astype(np.uint32)
    flo = ((bits >> np.uint32(9)) | np.uint32(0x3F800000)).view(np.float32)
    return np.maximum(np.float32(0), flo - np.float32(1.0)).reshape(B, T, NV)


def _pos_table():
    pos = np.arange(T, dtype=np.float32)[:, None]
    div = np.exp(np.arange(0, D, 2, dtype=np.float32) * (-np.log(10000.0) / D))
    pe = np.zeros((T, D), dtype=np.float32)
    pe[:, 0::2] = np.sin(pos * div)
    pe[:, 1::2] = np.cos(pos * div)
    return pe


@functools.lru_cache(maxsize=1)
def _constants():
    noise = _noise_constant()
    shuffle = np.argsort(noise, axis=-1, kind="stable").astype(np.int32)
    remain = shuffle[..., : NV // 2]
    masked = shuffle[..., NV // 2:]
    revert = np.argsort(shuffle, axis=-1, kind="stable").astype(np.int32)

    rem_flat = remain.reshape(B * T, NV // 2)
    u_all = np.arange(B * T, dtype=np.int32)
    src_lists = [u_all]
    dst_lists = [(u_all // T) * (NS_OUT * T) + (u_all % T)]
    for cval in range(NV):
        rows, cols = np.nonzero(rem_flat == cval)
        rows = rows.astype(np.int32)
        cols = cols.astype(np.int32)
        src_lists.append(rows)
        dst_lists.append((rows // T) * (NS_OUT * T) + (1 + cols) * T
                         + (rows % T))

    gsrc, gdst, chs = [], [], []
    for src, dst in zip(src_lists, dst_lists):
        n = src.shape[0]
        rs, rd = [], []
        for b in range(B):
            m = (src // T) == b
            sb, db = src[m], dst[m]
            k = int(np.searchsorted(sb % T, (b * T) // B))
            rs.append(np.roll(sb, -k))
            rd.append(np.roll(db, -k))
        src = np.concatenate(rs)
        dst = np.concatenate(rd)
        npad = -(-n // (NW * K)) * (NW * K)
        pad = npad - n
        if pad:
            pick = (np.arange(pad, dtype=np.int64) * n) // pad
            src = np.concatenate([src, src[pick]])
            dst = np.concatenate([dst, dst[pick]])
        ch = npad // (NW * K)
        gsrc.append(src)
        gdst.append(dst.reshape(NW, ch, K))
        chs.append(ch)

    chtot = sum(chs[1:])
    gsrc_packed = np.concatenate(
        [g.reshape(NW, c, K) for g, c in zip(gsrc[1:], chs[1:])], axis=1)
    gdst_packed = np.concatenate(gdst[1:], axis=1)
    assert gsrc_packed.shape == (NW, chtot, K)
    s0 = gsrc[0].reshape(NW, chs[0] * K)
    d0 = gdst[0].reshape(NW, chs[0] * K)
    assert (np.diff(s0, axis=1) == 1).all() and (np.diff(d0, axis=1) == 1).all()
    w = np.arange(NW)
    b0 = w // 8
    ts = (b0 * (T // B) + (w % 8) * (B * T // NW)) % T
    assert (s0[:, 0] == b0 * T + ts).all()
    assert (d0[:, 0] == b0 * (NS_OUT * T) + ts).all()

    sel = np.zeros((NS_OUT, B, T), dtype=np.float32)
    sel[1:] = np.moveaxis((remain == 0), -1, 0).astype(np.float32)

    return dict(
        masked=masked, revert=revert,
        pe=_pos_table(),
        gsrc=gsrc_packed, gdst=gdst_packed, chs=chs,
        sel=sel,
    )


def _sc_gather_fn(chs):
    mesh = plsc.VectorSubcoreMesh(core_axis_name="c", subcore_axis_name="s")
    scratch = []
    chtot = sum(chs[1:])
    choff = [None] + [sum(chs[1:c]) for c in range(1, 9)]
    scratch = [
        pltpu.VMEM((chtot, K), jnp.int32),
        pltpu.VMEM((chtot, K), jnp.int32),
        pltpu.VMEM((chtot, K), jnp.int32),
    ] + [
        pltpu.VMEM((D,), jnp.float32),
        pltpu.VMEM((2, K, D), jnp.float32),
        pltpu.VMEM((2, K, D), jnp.float32),
        pltpu.SemaphoreType.DMA((2,)),
        pltpu.SemaphoreType.DMA((2,)),
        pltpu.SemaphoreType.DMA,
    ]

    @functools.partial(
        pl.kernel,
        mesh=mesh,
        out_type=jax.ShapeDtypeStruct((NROWS_OUT, D), jnp.float32),
        scratch_types=scratch,
    )
    def body(*refs):
        xs = refs[0:9]
        pe_hbm = refs[9]
        mod_hbm = refs[10]
        gsrc = refs[11]
        gdst = refs[12]
        out = refs[13]
        isrc = refs[14]
        ipe = refs[15]
        idst = refs[16]
        modbuf = refs[17]
        xb, pb = refs[18], refs[19]
        sgx = refs[20]
        sgp = refs[21]
        ssc = refs[22]

        wid = lax.axis_index("s") * NC + lax.axis_index("c")
        pltpu.sync_copy(gsrc.at[wid], isrc)
        pltpu.sync_copy(gdst.at[wid], idst)

        def pe_idx_body(v, _):
            n = lax.shift_right_logical(v, 1)
            sl = pl.ds(lax.bitwise_and(v, 1) * LANES, LANES)
            ipe[n, sl] = lax.bitwise_and(isrc[n, sl], T - 1)
            return 0

        lax.fori_loop(0, chtot * (K // LANES), pe_idx_body, 0)

        b0 = lax.shift_right_logical(wid, 3)
        tstart = lax.bitwise_and(b0 * (T // B) + lax.bitwise_and(wid, 7)
                                 * (B * T // NW), T - 1)
        src0 = b0 * T + tstart
        dst0 = b0 * (NS_OUT * T) + tstart

        def issue_gather(c, chk, p):
            if c == 0:
                pltpu.async_copy(
                    xs[0].at[pl.ds(pl.multiple_of(src0 + chk * K, K), K)],
                    xb.at[p], sgx.at[p])
                pltpu.async_copy(
                    pe_hbm.at[pl.ds(pl.multiple_of(tstart + chk * K, K), K)],
                    pb.at[p], sgp.at[p])
            else:
                row = choff[c] + chk
                pltpu.async_copy(
                    xs[c].at[isrc.at[row]], xb.at[p], sgx.at[p])
                pltpu.async_copy(
                    pe_hbm.at[ipe.at[row]], pb.at[p], sgp.at[p])

        def wait_gather(c, p):
            pltpu.make_async_copy(xs[c].at[pl.ds(0, K)], xb.at[p],
                                  sgx.at[p]).wait()
            pltpu.make_async_copy(pe_hbm.at[pl.ds(0, K)], pb.at[p],
                                  sgp.at[p]).wait()

        def wait_scatter():
            pltpu.make_async_copy(xb.at[0], out.at[pl.ds(0, K)], ssc).wait()

        def compute(p):
            mods0 = tuple(modbuf[pl.ds(k * LANES, LANES)] for k in range(DV))

            def row_body(r, mods):
                for k in range(DV):
                    sl = pl.ds(k * LANES, LANES)
                    xb[p, r, sl] = xb[p, r, sl] + pb[p, r, sl] + mods[k]
                return mods

            lax.fori_loop(0, K, row_body, mods0)

        start_par = []
        s = 0
        for c in range(9):
            start_par.append(s)
            s = (s + chs[c]) % 2

        for c in range(9):
            CH = chs[c]
            pA = start_par[c]
            pltpu.sync_copy(mod_hbm.at[pl.ds(c * D, D)], modbuf)
            issue_gather(c, 0, pA)

            def chunk_body(n, _, c=c, CH=CH, pA=pA):
                p = lax.rem(pA + n, 2)
                wait_gather(c, p)
                if c == 0:
                    @pl.when(n > 0)
                    def _():
                        wait_scatter()
                else:
                    wait_scatter()

                @pl.when(n + 1 < CH)
                def _():
                    issue_gather(c, n + 1, 1 - p)

                compute(p)
                if c == 0:
                    pltpu.async_copy(
                        xb.at[p],
                        out.at[pl.ds(pl.multiple_of(dst0 + n * K, K), K)],
                        ssc)
                else:
                    pltpu.async_copy(xb.at[p],
                                     out.at[idst.at[choff[c] + n]], ssc)
                return 0

            lax.fori_loop(0, CH, chunk_body, 0)
        wait_scatter()

    return body


def _mask_body(t_ref, sel_ref, rm_ref, vm_ref):
    t = t_ref[...]
    tm1 = t[None] - 1.0
    rm_ref[...] = sel_ref[...] * tm1 + 1.0
    idx = lax.broadcasted_iota(jnp.int32, (9, B, T), 0)
    vm_ref[...] = jnp.where(idx == 1, t[None], jnp.float32(1.0))


def kernel(x_global, x_t0, x_t1, x_t2, x_t3, x_t4, x_t5, x_t6, x_t7,
           target_fcst_mask, mod_emb):
    C = _constants()
    xs = [jnp.reshape(a, (B * T, D)) for a in
          (x_global, x_t0, x_t1, x_t2, x_t3, x_t4, x_t5, x_t6, x_t7)]

    sc = _sc_gather_fn(tuple(C["chs"]))
    out = sc(*xs, jnp.asarray(C["pe"]), jnp.reshape(mod_emb, (9 * D,)),
             jnp.asarray(C["gsrc"]), jnp.asarray(C["gdst"]))
    remain_block = jnp.swapaxes(out.reshape(B, NS_OUT, T, D), 1, 2)

    rmask_p, vmask_p = pl.pallas_call(
        _mask_body,
        out_shape=[
            jax.ShapeDtypeStruct((NS_OUT, B, T), jnp.float32),
            jax.ShapeDtypeStruct((9, B, T), jnp.float32),
        ],
    )(target_fcst_mask, jnp.asarray(C["sel"]))
    rmask = jnp.transpose(rmask_p, (1, 2, 0))
    vmask = jnp.transpose(vmask_p, (1, 2, 0))

    return (remain_block, jnp.asarray(C["masked"]), jnp.asarray(C["revert"]),
            rmask, vmask)


# speedup vs baseline: 1.8584x; 1.8584x over previous
import functools

import jax
import jax.numpy as jnp
import numpy as np
from jax import lax
from jax.experimental import pallas as pl
from jax.experimental.pallas import tpu as pltpu
from jax.experimental.pallas import tpu_sc as plsc

B = 4
T = 2048
D = 768
NV = 8
NS_OUT = 5
NROWS_OUT = B * T * NS_OUT

NC = 2
NSUB = 16
NW = NC * NSUB
K = 32
LANES = 16
DV = D // LANES


def _rotl32(x, r):
    return ((x << np.uint32(r)) | (x >> np.uint32(32 - r))).astype(np.uint32)


def _threefry2x32(k0, k1, x0, x1):
    ks0 = np.uint32(k0)
    ks1 = np.uint32(k1)
    ks2 = np.uint32(ks0 ^ ks1 ^ np.uint32(0x1BD11BDA))
    x0 = (x0 + ks0).astype(np.uint32)
    x1 = (x1 + ks1).astype(np.uint32)
    rot0 = (13, 15, 26, 6)
    rot1 = (17, 29, 16, 24)
    ks = (ks0, ks1, ks2)
    for i in range(5):
        for r in rot0 if i % 2 == 0 else rot1:
            x0 = (x0 + x1).astype(np.uint32)
            x1 = _rotl32(x1, r)
            x1 = (x1 ^ x0).astype(np.uint32)
        x0 = (x0 + ks[(i + 1) % 3]).astype(np.uint32)
        x1 = (x1 + ks[(i + 2) % 3] + np.uint32(i + 1)).astype(np.uint32)
    return x0, x1


def _noise_constant():
    n = B * T * NV
    idx = np.arange(n, dtype=np.uint64)
    o0, o1 = _threefry2x32(0, 42, (idx >> np.uint64(32)).astype(np.uint32),
                           idx.astype(np.uint32))
    bits = (o0 ^ o1).astype(np.uint32)
    flo = ((bits >> np.uint32(9)) | np.uint32(0x3F800000)).view(np.float32)
    return np.maximum(np.float32(0), flo - np.float32(1.0)).reshape(B, T, NV)


def _pos_table():
    pos = np.arange(T, dtype=np.float32)[:, None]
    div = np.exp(np.arange(0, D, 2, dtype=np.float32) * (-np.log(10000.0) / D))
    pe = np.zeros((T, D), dtype=np.float32)
    pe[:, 0::2] = np.sin(pos * div)
    pe[:, 1::2] = np.cos(pos * div)
    return pe


@functools.lru_cache(maxsize=1)
def _constants():
    noise = _noise_constant()
    shuffle = np.argsort(noise, axis=-1, kind="stable").astype(np.int32)
    remain = shuffle[..., : NV // 2]
    masked = shuffle[..., NV // 2:]
    revert = np.argsort(shuffle, axis=-1, kind="stable").astype(np.int32)

    rem_flat = remain.reshape(B * T, NV // 2)
    u_all = np.arange(B * T, dtype=np.int32)
    src_lists = [u_all]
    dst_lists = [(u_all // T) * (NS_OUT * T) + (u_all % T)]
    for cval in range(NV):
        rows, cols = np.nonzero(rem_flat == cval)
        rows = rows.astype(np.int32)
        cols = cols.astype(np.int32)
        src_lists.append(rows)
        dst_lists.append((rows // T) * (NS_OUT * T) + (1 + cols) * T
                         + (rows % T))

    gsrc, gdst, chs = [], [], []
    for src, dst in zip(src_lists, dst_lists):
        n = src.shape[0]
        rs, rd = [], []
        for b in range(B):
            m = (src // T) == b
            sb, db = src[m], dst[m]
            k = int(np.searchsorted(sb % T, (b * T) // B))
            rs.append(np.roll(sb, -k))
            rd.append(np.roll(db, -k))
        src = np.concatenate(rs)
        dst = np.concatenate(rd)
        npad = -(-n // (NW * K)) * (NW * K)
        pad = npad - n
        if pad:
            pick = (np.arange(pad, dtype=np.int64) * n) // pad
            src = np.concatenate([src, src[pick]])
            dst = np.concatenate([dst, dst[pick]])
        ch = npad // (NW * K)
        gsrc.append(src)
        gdst.append(dst.reshape(NW, ch, K))
        chs.append(ch)

    chtot = sum(chs)
    gsrc_packed = np.concatenate(
        [g.reshape(NW, c, K) for g, c in zip(gsrc, chs)], axis=1)
    gdst_packed = np.concatenate(gdst, axis=1)
    assert gsrc_packed.shape == (NW, chtot, K)

    sel = np.zeros((NS_OUT, B, T), dtype=np.float32)
    sel[1:] = np.moveaxis((remain == 0), -1, 0).astype(np.float32)

    return dict(
        masked=masked, revert=revert,
        pe=_pos_table(),
        gsrc=gsrc_packed, gdst=gdst_packed, chs=chs,
        sel=sel,
    )


def _sc_gather_fn(chs):
    mesh = plsc.VectorSubcoreMesh(core_axis_name="c", subcore_axis_name="s")
    scratch = []
    chtot = sum(chs)
    choff = [sum(chs[:c]) for c in range(9)]
    scratch = [
        pltpu.VMEM((chtot, K), jnp.int32),
        pltpu.VMEM((chtot, K), jnp.int32),
        pltpu.VMEM((chtot, K), jnp.int32),
    ] + [
        pltpu.VMEM((D,), jnp.float32),
        pltpu.VMEM((2, K, D), jnp.float32),
        pltpu.VMEM((2, K, D), jnp.float32),
        pltpu.SemaphoreType.DMA((2,)),
        pltpu.SemaphoreType.DMA((2,)),
        pltpu.SemaphoreType.DMA,
    ]

    @functools.partial(
        pl.kernel,
        mesh=mesh,
        out_type=jax.ShapeDtypeStruct((NROWS_OUT, D), jnp.float32),
        scratch_types=scratch,
    )
    def body(*refs):
        xs = refs[0:9]
        pe_hbm = refs[9]
        mod_hbm = refs[10]
        gsrc = refs[11]
        gdst = refs[12]
        out = refs[13]
        isrc = refs[14]
        ipe = refs[15]
        idst = refs[16]
        modbuf = refs[17]
        xb, pb = refs[18], refs[19]
        sgx = refs[20]
        sgp = refs[21]
        ssc = refs[22]

        wid = lax.axis_index("s") * NC + lax.axis_index("c")
        pltpu.sync_copy(gsrc.at[wid], isrc)
        pltpu.sync_copy(gdst.at[wid], idst)

        def pe_idx_body(v, _):
            n = lax.shift_right_logical(v, 1)
            sl = pl.ds(lax.bitwise_and(v, 1) * LANES, LANES)
            ipe[n, sl] = lax.bitwise_and(isrc[n, sl], T - 1)
            return 0

        lax.fori_loop(0, chtot * (K // LANES), pe_idx_body, 0)

        def issue_gather(c, chk, p):
            row = choff[c] + chk
            pltpu.async_copy(
                xs[c].at[isrc.at[row]], xb.at[p], sgx.at[p])
            pltpu.async_copy(
                pe_hbm.at[ipe.at[row]], pb.at[p], sgp.at[p])

        def wait_gather(c, p):
            pltpu.make_async_copy(xs[c].at[pl.ds(0, K)], xb.at[p],
                                  sgx.at[p]).wait()
            pltpu.make_async_copy(pe_hbm.at[pl.ds(0, K)], pb.at[p],
                                  sgp.at[p]).wait()

        def wait_scatter():
            pltpu.make_async_copy(xb.at[0], out.at[pl.ds(0, K)], ssc).wait()

        def compute(p):
            mods0 = tuple(modbuf[pl.ds(k * LANES, LANES)] for k in range(DV))

            def row_body(r, mods):
                for k in range(DV):
                    sl = pl.ds(k * LANES, LANES)
                    xb[p, r, sl] = xb[p, r, sl] + pb[p, r, sl] + mods[k]
                return mods

            lax.fori_loop(0, K, row_body, mods0)

        start_par = []
        s = 0
        for c in range(9):
            start_par.append(s)
            s = (s + chs[c]) % 2

        for c in range(9):
            CH = chs[c]
            pA = start_par[c]
            pltpu.sync_copy(mod_hbm.at[pl.ds(c * D, D)], modbuf)
            issue_gather(c, 0, pA)

            def chunk_body(n, _, c=c, CH=CH, pA=pA):
                p = lax.rem(pA + n, 2)
                wait_gather(c, p)
                if c == 0:
                    @pl.when(n > 0)
                    def _():
                        wait_scatter()
                else:
                    wait_scatter()

                @pl.when(n + 1 < CH)
                def _():
                    issue_gather(c, n + 1, 1 - p)

                compute(p)
                pltpu.async_copy(xb.at[p],
                                 out.at[idst.at[choff[c] + n]], ssc)
                return 0

            lax.fori_loop(0, CH, chunk_body, 0)
        wait_scatter()

    return body


def _mask_body(t_ref, sel_ref, rm_ref, vm_ref):
    t = t_ref[...]
    tm1 = t[None] - 1.0
    rm_ref[...] = sel_ref[...] * tm1 + 1.0
    idx = lax.broadcasted_iota(jnp.int32, (9, B, T), 0)
    vm_ref[...] = jnp.where(idx == 1, t[None], jnp.float32(1.0))


def kernel(x_global, x_t0, x_t1, x_t2, x_t3, x_t4, x_t5, x_t6, x_t7,
           target_fcst_mask, mod_emb):
    C = _constants()
    xs = [jnp.reshape(a, (B * T, D)) for a in
          (x_global, x_t0, x_t1, x_t2, x_t3, x_t4, x_t5, x_t6, x_t7)]

    sc = _sc_gather_fn(tuple(C["chs"]))
    out = sc(*xs, jnp.asarray(C["pe"]), jnp.reshape(mod_emb, (9 * D,)),
             jnp.asarray(C["gsrc"]), jnp.asarray(C["gdst"]))
    remain_block = jnp.swapaxes(out.reshape(B, NS_OUT, T, D), 1, 2)

    rmask_p, vmask_p = pl.pallas_call(
        _mask_body,
        out_shape=[
            jax.ShapeDtypeStruct((NS_OUT, B, T), jnp.float32),
            jax.ShapeDtypeStruct((9, B, T), jnp.float32),
        ],
    )(target_fcst_mask, jnp.asarray(C["sel"]))
    rmask = jnp.transpose(rmask_p, (1, 2, 0))
    vmask = jnp.transpose(vmask_p, (1, 2, 0))

    return (remain_block, jnp.asarray(C["masked"]), jnp.asarray(C["revert"]),
            rmask, vmask)
